# baseline (speedup 1.0000x reference)
import jax
import jax.numpy as jnp
from jax.experimental import pallas as pl
from jax.experimental.pallas import tpu as pltpu


def _mlp_step(x_hbm, gup_hbm, dp_hbm, o_hbm,
              xbuf, obuf, wgu_stage, wdp_stage, wgu16, wdp16,
              lsem, ssem, wgsem, wdsem):
    ei = pl.program_id(0)
    ne = pl.num_programs(0)
    slot = jax.lax.rem(ei, 2)
    nslot = jax.lax.rem(ei + 1, 2)

    hh2 = gup_hbm.shape[1] // 2

    def xloads(kk, sl):
        return [pltpu.make_async_copy(
            x_hbm.at[bj, :, kk, :], xbuf.at[sl, bj], lsem.at[sl, bj])
            for bj in range(x_hbm.shape[0])]

    def wguloads(kk, sl):
        return [pltpu.make_async_copy(
            gup_hbm.at[kk, hj * hh2:(hj + 1) * hh2, :],
            wgu_stage.at[sl, hj * hh2:(hj + 1) * hh2, :],
            wgsem.at[sl, hj]) for hj in range(2)]

    def wdpload(kk, sl):
        return pltpu.make_async_copy(
            dp_hbm.at[kk], wdp_stage.at[sl], wdsem.at[sl])

    @pl.when(ei == 0)
    def _():
        for c in xloads(ei, slot) + wguloads(ei, slot):
            c.start()
        wdpload(ei, slot).start()

    @pl.when(ei + 1 < ne)
    def _():
        for c in xloads(ei + 1, nslot) + wguloads(ei + 1, nslot):
            c.start()
        wdpload(ei + 1, nslot).start()

    for c in wguloads(ei, slot):
        c.wait()
    wdpload(ei, slot).wait()
    wgu16[...] = wgu_stage[slot].astype(jnp.bfloat16)
    wdp16[...] = wdp_stage[slot].astype(jnp.bfloat16)

    for c in xloads(ei, slot):
        c.wait()

    bb, gg, hh = xbuf.shape[1], xbuf.shape[2], xbuf.shape[3]
    ih = wdp16.shape[0]
    xe = xbuf[slot].reshape(bb * gg, hh).astype(jnp.bfloat16)
    gu = jnp.dot(xe, wgu16[...], preferred_element_type=jnp.float32)
    inter = (jax.nn.silu(gu[:, :ih]) * gu[:, ih:]).astype(jnp.bfloat16)

    def stores(sl):
        return [pltpu.make_async_copy(
            obuf.at[sl, bj], o_hbm.at[bj, :, ei, :], ssem.at[sl, bj])
            for bj in range(o_hbm.shape[0])]

    @pl.when(ei >= 2)
    def _():
        for c in stores(slot):
            c.wait()

    obuf[slot] = jnp.dot(inter, wdp16[...],
                         preferred_element_type=jnp.float32).reshape(bb, gg, hh)
    for c in stores(slot):
        c.start()

    @pl.when(ei == ne - 1)
    def _():
        for c in stores(slot) + stores(nslot):
            c.wait()


def kernel(x, gate_up_proj, down_proj):
    b, n, h = x.shape
    e, _, i2 = gate_up_proj.shape
    i = i2 // 2
    g = n // e
    x4 = x.reshape(b, g, e, h)
    out4 = pl.pallas_call(
        _mlp_step,
        grid=(e,),
        in_specs=[
            pl.BlockSpec(memory_space=pl.ANY),
            pl.BlockSpec(memory_space=pl.ANY),
            pl.BlockSpec(memory_space=pl.ANY),
        ],
        out_specs=pl.BlockSpec(memory_space=pl.ANY),
        out_shape=jax.ShapeDtypeStruct((b, g, e, h), jnp.float32),
        scratch_shapes=[
            pltpu.VMEM((2, b, g, h), jnp.float32),
            pltpu.VMEM((2, b, g, h), jnp.float32),
            pltpu.VMEM((2, h, i2), jnp.float32),
            pltpu.VMEM((2, i, h), jnp.float32),
            pltpu.VMEM((h, i2), jnp.bfloat16),
            pltpu.VMEM((i, h), jnp.bfloat16),
            pltpu.SemaphoreType.DMA((2, 2)),
            pltpu.SemaphoreType.DMA((2, 2)),
            pltpu.SemaphoreType.DMA((2, 2)),
            pltpu.SemaphoreType.DMA((2,)),
        ],
    )(x4, gate_up_proj, down_proj)
    return out4.reshape(b, n, h)
